# baseline (speedup 1.0000x reference)
import functools

import jax
import jax.numpy as jnp
from jax import lax
from jax.experimental import pallas as pl
from jax.experimental.pallas import tpu as pltpu
from jax.experimental.pallas import tpu_sc as plsc

NC = 2
NS = 16
NW = NC * NS

CH_SEQS = 2
NG = 5
L = 16


def kernel(x, token_table, pos_table):
    batch, seq = x.shape
    vocab, embed = token_table.shape
    assert embed % L == 0
    nrows = batch * seq
    bpw = nrows // NW
    assert bpw * NW == nrows and bpw % seq == 0
    ch = CH_SEQS * seq
    gr = ch // NG
    assert gr * NG == ch and gr % 8 == 0 and gr <= 128
    nit = bpw // ch
    assert nit * ch == bpw

    x_flat = x.reshape(nrows).astype(jnp.int32)

    mesh = plsc.VectorSubcoreMesh(core_axis_name="c", subcore_axis_name="s")

    @functools.partial(
        pl.kernel,
        mesh=mesh,
        compiler_params=pltpu.CompilerParams(use_tc_tiling_on_sc=False),
        out_type=jax.ShapeDtypeStruct((nrows, embed), jnp.float32),
        scratch_types=[
            pltpu.VMEM((ch,), jnp.int32),
            pltpu.VMEM((ch, embed), jnp.float32),
            pltpu.VMEM((seq, embed), jnp.float32),
            pltpu.SemaphoreType.DMA,
        ],
    )
    def emb(x_hbm, tok_hbm, pos_hbm, out_hbm, idx_v, rows_v, pos_v, sem):
        wid = lax.axis_index("s") * NC + lax.axis_index("c")
        base = wid * bpw
        pltpu.sync_copy(pos_hbm, pos_v)

        def chunk_body(g, carry):
            row0 = base + g * ch
            pltpu.sync_copy(x_hbm.at[pl.ds(row0, ch)], idx_v)
            copies = [
                pltpu.async_copy(
                    tok_hbm.at[idx_v.at[pl.ds(j * gr, gr)]],
                    rows_v.at[pl.ds(j * gr, gr)],
                    sem,
                )
                for j in range(NG)
            ]
            for c in copies:
                c.wait()

            def pos_body(p, c2):
                for q in range(embed // L):
                    pv = pos_v[p, pl.ds(q * L, L)]
                    for s in range(CH_SEQS):
                        plsc.addupdate(rows_v.at[p + s * seq, pl.ds(q * L, L)], pv)
                return c2

            lax.fori_loop(0, seq, pos_body, 0)
            pltpu.sync_copy(rows_v, out_hbm.at[pl.ds(row0, ch)])
            return carry

        lax.fori_loop(0, nit, chunk_body, 0)

    out = emb(x_flat, token_table, pos_table)
    return out.reshape(batch, seq, embed)


# speedup vs baseline: 1.1276x; 1.1276x over previous
import functools

import jax
import jax.numpy as jnp
from jax import lax
from jax.experimental import pallas as pl
from jax.experimental.pallas import tpu as pltpu
from jax.experimental.pallas import tpu_sc as plsc

NC = 2
NS = 16
NW = NC * NS

CH_SEQS = 2
NG = 5
NBUF = 3
L = 16


def kernel(x, token_table, pos_table):
    batch, seq = x.shape
    vocab, embed = token_table.shape
    assert embed % L == 0
    nrows = batch * seq
    bpw = nrows // NW
    assert bpw * NW == nrows and bpw % seq == 0
    ch = CH_SEQS * seq
    gr = ch // NG
    assert gr * NG == ch and gr % 8 == 0 and gr <= 128
    nit = bpw // ch
    assert nit * ch == bpw and nit >= NBUF
    nmain = nit - 1
    assert nmain % NBUF == 0

    x_flat = x.reshape(nrows).astype(jnp.int32)

    mesh = plsc.VectorSubcoreMesh(core_axis_name="c", subcore_axis_name="s")

    @functools.partial(
        pl.kernel,
        mesh=mesh,
        compiler_params=pltpu.CompilerParams(use_tc_tiling_on_sc=False),
        out_type=jax.ShapeDtypeStruct((nrows, embed), jnp.float32),
        scratch_types=(
            [pltpu.VMEM((bpw,), jnp.int32),
             pltpu.VMEM((seq, embed), jnp.float32)]
            + [pltpu.VMEM((ch, embed), jnp.float32) for _ in range(NBUF)]
            + [pltpu.SemaphoreType.DMA for _ in range(2 * NBUF)]
        ),
    )
    def emb(x_hbm, tok_hbm, pos_hbm, out_hbm, idx_v, pos_v, *bufs_sems):
        rows = bufs_sems[:NBUF]
        gsem = bufs_sems[NBUF:2 * NBUF]
        ssem = bufs_sems[2 * NBUF:]
        wid = lax.axis_index("s") * NC + lax.axis_index("c")
        base = wid * bpw

        pltpu.sync_copy(pos_hbm, pos_v)
        pltpu.sync_copy(x_hbm.at[pl.ds(base, bpw)], idx_v)

        def fire(g, b):
            for j in range(NG):
                pltpu.async_copy(
                    tok_hbm.at[idx_v.at[pl.ds(g * ch + j * gr, gr)]],
                    rows[b].at[pl.ds(j * gr, gr)],
                    gsem[b],
                )

        def drain_gathers(b):
            pltpu.make_async_copy(
                tok_hbm.at[pl.ds(0, ch)],
                rows[b], gsem[b],
            ).wait()

        def wait_store(g, b):
            pltpu.make_async_copy(
                rows[b], out_hbm.at[pl.ds(base + g * ch, ch)], ssem[b],
            ).wait()

        def pos_add(b):
            def body(p, c):
                for q in range(embed // L):
                    pv = pos_v[p, pl.ds(q * L, L)]
                    for s in range(CH_SEQS):
                        plsc.addupdate(rows[b].at[p + s * seq, pl.ds(q * L, L)], pv)
                return c
            lax.fori_loop(0, seq, body, 0)

        def slot(g, b, prefetch):
            drain_gathers(b)
            pos_add(b)
            pltpu.async_copy(rows[b], out_hbm.at[pl.ds(base + g * ch, ch)], ssem[b])
            if prefetch:
                bpre = (b + NBUF - 1) % NBUF

                @pl.when(g + (NBUF - 1) < nit)
                def _():
                    @pl.when(g >= 1)
                    def _():
                        wait_store(g - 1, bpre)

                    fire(g + NBUF - 1, bpre)

        for b in range(NBUF - 1):
            fire(b, b)

        def outer(o, c):
            for b in range(NBUF):
                slot(o * NBUF + b, b, prefetch=True)
            return c
        lax.fori_loop(0, nmain // NBUF, outer, 0)

        slot(nit - 1, (nit - 1) % NBUF, prefetch=False)
        for g in range(nit - NBUF, nit):
            wait_store(g, g % NBUF)

    out = emb(x_flat, token_table, pos_table)
    return out.reshape(batch, seq, embed)
